# baseline (speedup 1.0000x reference)
import jax
import jax.numpy as jnp
from jax.experimental import pallas as pl

_HEAD_DIMS = (14, 2, 4, 14, 2)


def _fused_head_kernel(x_ref, w_ref, b_ref, box_ref, cls_ref, dir_ref,
                       var_ref, iou_ref):
    TH = box_ref.shape[1]
    W = box_ref.shape[2]
    xt = x_ref[0]
    acc = jax.lax.dot_general(
        w_ref[...], xt, (((1,), (0,)), ((), ())),
        preferred_element_type=jnp.float32)
    accT = acc.T + b_ref[...]
    box_ref[0] = accT[:, 0:14].reshape(TH, W, 14)
    cls_ref[0] = accT[:, 14:16].reshape(TH, W, 2)
    dir_ref[0] = accT[:, 16:20].reshape(TH, W, 4)
    var_ref[0] = accT[:, 20:34].reshape(TH, W, 14)
    iou_ref[0] = accT[:, 34:36].reshape(TH, W, 2)


def kernel(x, W_box, b_box, W_cls, b_cls, W_iou, W_dir, b_dir, W_var, b_var):
    B, C, H, W = x.shape
    HW = H * W
    Wc = jnp.concatenate([W_box, W_cls, W_dir, W_var, W_iou], axis=0)
    bc = jnp.concatenate(
        [b_box, b_cls, b_dir, b_var, jnp.zeros((2,), x.dtype)], axis=0)
    bc2 = bc.reshape(1, 36)

    TH = 16
    MT = TH * W
    nH = H // TH
    x2 = x.reshape(B, C, HW)

    outs = pl.pallas_call(
        _fused_head_kernel,
        grid=(B, nH),
        in_specs=[
            pl.BlockSpec((1, C, MT), lambda b, h: (b, 0, h)),
            pl.BlockSpec((36, C), lambda b, h: (0, 0)),
            pl.BlockSpec((1, 36), lambda b, h: (0, 0)),
        ],
        out_specs=[
            pl.BlockSpec((1, TH, W, o), lambda b, h: (b, h, 0, 0))
            for o in _HEAD_DIMS
        ],
        out_shape=[
            jax.ShapeDtypeStruct((B, H, W, o), x.dtype) for o in _HEAD_DIMS
        ],
    )(x2, Wc, bc2)

    return tuple(outs)


# speedup vs baseline: 7.1808x; 7.1808x over previous
import jax
import jax.numpy as jnp
from jax.experimental import pallas as pl

_HEAD_DIMS = (14, 2, 4, 14, 2)
_TH = 8


def _fused_head_kernel(x_ref, m_ref, b_ref, box_ref, cls_ref, dir_ref,
                       var_ref, iou_ref):
    C, TH, W = x_ref.shape[1:]
    xs = x_ref[0].reshape(C * TH, W)
    acc = jax.lax.dot_general(
        m_ref[...], xs, (((1,), (0,)), ((), ())),
        preferred_element_type=jnp.float32)
    acc = acc + b_ref[...][:, 0:1]
    box_ref[0] = acc[0:112].reshape(14, TH, W)
    cls_ref[0] = acc[112:128].reshape(2, TH, W)
    dir_ref[0] = acc[128:160].reshape(4, TH, W)
    var_ref[0] = acc[160:272].reshape(14, TH, W)
    iou_ref[0] = acc[272:288].reshape(2, TH, W)


def kernel(x, W_box, b_box, W_cls, b_cls, W_iou, W_dir, b_dir, W_var, b_var):
    B, C, H, W = x.shape
    TH = _TH
    nH = H // TH
    Wc = jnp.concatenate([W_box, W_cls, W_dir, W_var, W_iou], axis=0)
    bc = jnp.concatenate(
        [b_box, b_cls, b_dir, b_var, jnp.zeros((2,), x.dtype)], axis=0)
    eye = jnp.eye(TH, dtype=x.dtype)
    M = (Wc[:, None, :, None] * eye[None, :, None, :]).reshape(36 * TH, C * TH)
    b2 = jnp.tile(jnp.repeat(bc, TH)[:, None], (1, 128))

    outs = pl.pallas_call(
        _fused_head_kernel,
        grid=(B, nH),
        in_specs=[
            pl.BlockSpec((1, C, TH, W), lambda b, h: (b, 0, h, 0)),
            pl.BlockSpec((36 * TH, C * TH), lambda b, h: (0, 0)),
            pl.BlockSpec((36 * TH, 128), lambda b, h: (0, 0)),
        ],
        out_specs=[
            pl.BlockSpec((1, o, TH, W), lambda b, h: (b, 0, h, 0))
            for o in _HEAD_DIMS
        ],
        out_shape=[
            jax.ShapeDtypeStruct((B, o, H, W), x.dtype) for o in _HEAD_DIMS
        ],
    )(x, M, b2)

    return tuple(jnp.transpose(o, (0, 2, 3, 1)) for o in outs)
